# baseline (speedup 1.0000x reference)
import functools

import jax
import jax.numpy as jnp
from jax.experimental import pallas as pl
from jax.experimental.pallas import tpu as pltpu


def _moe_body(x_ref, wg_ref, bg_ref, w1_ref, b1_ref, w2_ref, b2_ref,
              out_ref, gate_ref, coeff_ref, *, bt: int, n_exp: int):
    e = pl.program_id(0)
    b = pl.program_id(1)

    @pl.when((e == 0) & (b == 0))
    def _gate():
        x = x_ref[...]
        g = jnp.dot(x, wg_ref[...], preferred_element_type=jnp.float32)
        g = g + bg_ref[...]
        gate_ref[...] = g
        idx = jax.lax.broadcasted_iota(jnp.int32, g.shape, 1)
        v0 = jnp.max(g, axis=1, keepdims=True)
        s0 = jnp.min(jnp.where(g == v0, idx, n_exp), axis=1, keepdims=True)
        g2 = jnp.where(idx == s0, -jnp.inf, g)
        v1 = jnp.max(g2, axis=1, keepdims=True)
        s1 = jnp.min(jnp.where(g2 == v1, idx, n_exp), axis=1, keepdims=True)
        ed = jnp.exp(v1 - v0)
        w0 = 1.0 / (1.0 + ed)
        w1 = ed / (1.0 + ed)
        coeff_ref[...] = (jnp.where(idx == s0, w0, 0.0)
                          + jnp.where(idx == s1, w1, 0.0))
        out_ref[...] = x

    xb = x_ref[pl.ds(b * bt, bt), :].astype(jnp.bfloat16)
    h = jnp.dot(xb, w1_ref[0], preferred_element_type=jnp.float32)
    h = jnp.maximum(h + b1_ref[0], 0.0).astype(jnp.bfloat16)
    y = jnp.dot(h, w2_ref[0], preferred_element_type=jnp.float32)
    y = y + b2_ref[0]
    cb = coeff_ref[pl.ds(b * bt, bt), :]
    eidx = jax.lax.broadcasted_iota(jnp.int32, cb.shape, 1)
    c = jnp.sum(jnp.where(eidx == e, cb, 0.0), axis=1, keepdims=True)
    out_ref[pl.ds(b * bt, bt), :] += c * y


def kernel(inputs_raw, Wg, bg, W1, b1, W2, b2):
    ishape = inputs_raw.shape
    d = ishape[-1]
    t = inputs_raw.size // d
    n_exp, dff = W1.shape[0], W1.shape[2]
    bt = min(256, t)
    nb = t // bt

    x = inputs_raw.reshape(t, d)
    body = functools.partial(_moe_body, bt=bt, n_exp=n_exp)
    out, gate = pl.pallas_call(
        body,
        grid=(n_exp, nb),
        in_specs=[
            pl.BlockSpec((t, d), lambda e, b: (0, 0)),
            pl.BlockSpec((d, n_exp), lambda e, b: (0, 0)),
            pl.BlockSpec((1, n_exp), lambda e, b: (0, 0)),
            pl.BlockSpec((1, d, dff), lambda e, b: (e, 0, 0)),
            pl.BlockSpec((1, 1, dff), lambda e, b: (e, 0, 0)),
            pl.BlockSpec((1, dff, d), lambda e, b: (e, 0, 0)),
            pl.BlockSpec((1, 1, d), lambda e, b: (e, 0, 0)),
        ],
        out_specs=[
            pl.BlockSpec((t, d), lambda e, b: (0, 0)),
            pl.BlockSpec((t, n_exp), lambda e, b: (0, 0)),
        ],
        out_shape=[
            jax.ShapeDtypeStruct((t, d), jnp.float32),
            jax.ShapeDtypeStruct((t, n_exp), jnp.float32),
        ],
        scratch_shapes=[pltpu.VMEM((t, n_exp), jnp.float32)],
        compiler_params=pltpu.CompilerParams(
            dimension_semantics=("arbitrary", "arbitrary"),
        ),
    )(x, Wg, bg.reshape(1, n_exp), W1.astype(jnp.bfloat16),
      b1.reshape(n_exp, 1, dff), W2.astype(jnp.bfloat16),
      b2.reshape(n_exp, 1, d))
    return out.reshape(ishape), gate


# speedup vs baseline: 1.2114x; 1.2114x over previous
import functools

import jax
import jax.numpy as jnp
from jax.experimental import pallas as pl
from jax.experimental.pallas import tpu as pltpu


def _moe_body(x_ref, wg_ref, bg_ref, w1_ref, b1_ref, w2_ref, b2_ref,
              out_ref, gate_ref, coeff_ref, *, bt: int, n_exp: int):
    e = pl.program_id(0)
    b = pl.program_id(1)

    @pl.when((e == 0) & (b == 0))
    def _gate():
        x = x_ref[...]
        g = jnp.dot(x, wg_ref[...], preferred_element_type=jnp.float32)
        g = g + bg_ref[...]
        gate_ref[...] = g
        idx = jax.lax.broadcasted_iota(jnp.int32, g.shape, 1)
        v0 = jnp.max(g, axis=1, keepdims=True)
        s0 = jnp.min(jnp.where(g == v0, idx, n_exp), axis=1, keepdims=True)
        g2 = jnp.where(idx == s0, -jnp.inf, g)
        v1 = jnp.max(g2, axis=1, keepdims=True)
        s1 = jnp.min(jnp.where(g2 == v1, idx, n_exp), axis=1, keepdims=True)
        ed = jnp.exp(v1 - v0)
        w0 = 1.0 / (1.0 + ed)
        w1 = ed / (1.0 + ed)
        coeff_ref[...] = (jnp.where(idx == s0, w0, 0.0)
                          + jnp.where(idx == s1, w1, 0.0))
        out_ref[...] = x

    xb = x_ref[pl.ds(b * bt, bt), :]
    h = jnp.dot(xb, w1_ref[0], preferred_element_type=jnp.float32,
                precision=jax.lax.Precision.DEFAULT)
    h = jnp.maximum(h + b1_ref[0], 0.0)
    y = jnp.dot(h, w2_ref[0], preferred_element_type=jnp.float32,
                precision=jax.lax.Precision.DEFAULT)
    y = y + b2_ref[0]
    cb = coeff_ref[pl.ds(b * bt, bt), :]
    eidx = jax.lax.broadcasted_iota(jnp.int32, cb.shape, 1)
    c = jnp.sum(jnp.where(eidx == e, cb, 0.0), axis=1, keepdims=True)
    out_ref[pl.ds(b * bt, bt), :] += c * y


def kernel(inputs_raw, Wg, bg, W1, b1, W2, b2):
    ishape = inputs_raw.shape
    d = ishape[-1]
    t = inputs_raw.size // d
    n_exp, dff = W1.shape[0], W1.shape[2]
    bt = min(256, t)
    nb = t // bt

    x = inputs_raw.reshape(t, d)
    body = functools.partial(_moe_body, bt=bt, n_exp=n_exp)
    out, gate = pl.pallas_call(
        body,
        grid=(n_exp, nb),
        in_specs=[
            pl.BlockSpec((t, d), lambda e, b: (0, 0)),
            pl.BlockSpec((d, n_exp), lambda e, b: (0, 0)),
            pl.BlockSpec((1, n_exp), lambda e, b: (0, 0)),
            pl.BlockSpec((1, d, dff), lambda e, b: (e, 0, 0)),
            pl.BlockSpec((1, 1, dff), lambda e, b: (e, 0, 0)),
            pl.BlockSpec((1, dff, d), lambda e, b: (e, 0, 0)),
            pl.BlockSpec((1, 1, d), lambda e, b: (e, 0, 0)),
        ],
        out_specs=[
            pl.BlockSpec((t, d), lambda e, b: (0, 0)),
            pl.BlockSpec((t, n_exp), lambda e, b: (0, 0)),
        ],
        out_shape=[
            jax.ShapeDtypeStruct((t, d), jnp.float32),
            jax.ShapeDtypeStruct((t, n_exp), jnp.float32),
        ],
        scratch_shapes=[pltpu.VMEM((t, n_exp), jnp.float32)],
        compiler_params=pltpu.CompilerParams(
            dimension_semantics=("arbitrary", "arbitrary"),
        ),
    )(x, Wg, bg.reshape(1, n_exp), W1, b1.reshape(n_exp, 1, dff), W2,
      b2.reshape(n_exp, 1, d))
    return out.reshape(ishape), gate
